# Initial kernel scaffold; baseline (speedup 1.0000x reference)
import jax
import jax.numpy as jnp
from jax.experimental import pallas as pl


def kernel(q, k, v, mask, weights1_real, weights1_imag):
    raise NotImplementedError("write your pallas kernel here")


# speedup vs baseline: 5.2053x; 5.2053x over previous
import math

import jax
import jax.numpy as jnp
from jax.experimental import pallas as pl

_MODES = 8


def _fca_kernel(q_ref, wr_ref, wi_ref, bf_ref, bi_ref, out_ref):
    x = q_ref[0]
    bf = bf_ref[...]
    X = jax.lax.dot_general(bf, x, (((1,), (0,)), ((), ())),
                            precision=jax.lax.Precision.HIGHEST,
                            preferred_element_type=jnp.float32)
    yr_rows = []
    yi_rows = []
    for m in range(_MODES):
        xr = X[m:m + 1, :]
        xi = X[_MODES + m:_MODES + m + 1, :]
        wr = wr_ref[m]
        wi = wi_ref[m]
        dot = lambda a, b: jax.lax.dot_general(
            a, b, (((1,), (0,)), ((), ())),
            precision=jax.lax.Precision.HIGHEST,
            preferred_element_type=jnp.float32)
        yr_rows.append(dot(xr, wr) - dot(xi, wi))
        yi_rows.append(dot(xr, wi) + dot(xi, wr))
    coeffs = jnp.concatenate(yr_rows + yi_rows, axis=0)
    bi = bi_ref[...]
    out = jax.lax.dot_general(bi, coeffs, (((1,), (0,)), ((), ())),
                              precision=jax.lax.Precision.HIGHEST,
                              preferred_element_type=jnp.float32)
    out_ref[0] = out


def kernel(q, k, v, mask, weights1_real, weights1_imag):
    del k, v, mask
    B, L, H, E = q.shape
    M = _MODES

    m = jnp.arange(M, dtype=jnp.int32)
    l = jnp.arange(L, dtype=jnp.int32)
    phase = (m[:, None] * l[None, :]) % L
    ang = phase.astype(jnp.float32) * jnp.float32(2.0 * math.pi / L)
    c = jnp.cos(ang)
    s = jnp.sin(ang)
    bf = jnp.concatenate([c, -s], axis=0)

    sc = jnp.full((M, 1), 2.0 / L, dtype=jnp.float32).at[0, 0].set(1.0 / L)
    si = jnp.full((M, 1), -1.0 / L, dtype=jnp.float32).at[0, 0].set(0.0)
    bi = jnp.concatenate([sc * c, si * s], axis=0).T

    q3 = q.reshape(B, L, H * E)
    out = pl.pallas_call(
        _fca_kernel,
        grid=(B * H,),
        in_specs=[
            pl.BlockSpec((1, L, E), lambda i: (i // H, 0, i % H)),
            pl.BlockSpec((M, E, E), lambda i: (0, 0, 0)),
            pl.BlockSpec((M, E, E), lambda i: (0, 0, 0)),
            pl.BlockSpec((2 * M, L), lambda i: (0, 0)),
            pl.BlockSpec((L, 2 * M), lambda i: (0, 0)),
        ],
        out_specs=pl.BlockSpec((1, L, E), lambda i: (i // H, 0, i % H)),
        out_shape=jax.ShapeDtypeStruct((B, L, H * E), jnp.float32),
    )(q3, weights1_real, weights1_imag, bf, bi)
    return out.reshape(B, L, H, E)


# speedup vs baseline: 9.3179x; 1.7901x over previous
import math

import jax
import jax.numpy as jnp
from jax.experimental import pallas as pl
from jax.experimental.pallas import tpu as pltpu

_MODES = 8
_NC = 2 * _MODES


def _split_bf16(a):
    hi = a.astype(jnp.bfloat16).astype(jnp.float32)
    return hi, a - hi


def _fca_kernel(q_ref, wr_ref, wi_ref, bf_ref, bic_ref, out_ref, x_scr, c_scr):
    i = pl.program_id(0)
    np_ = x_scr.shape[0]

    @pl.when(i < np_)
    def _forward():
        x = q_ref[0]
        bf2 = bf_ref[...]
        x2 = jax.lax.dot_general(bf2, x, (((1,), (0,)), ((), ())),
                                 preferred_element_type=jnp.float32)
        x_scr[pl.ds(i, 1)] = (x2[:_NC] + x2[_NC:])[None]

    @pl.when(i == np_)
    def _mix():
        dot = lambda a, b: jax.lax.dot_general(
            a, b, (((1,), (0,)), ((), ())),
            preferred_element_type=jnp.float32)
        for m in range(_MODES):
            s = jnp.concatenate([x_scr[:, m, :], x_scr[:, _MODES + m, :]],
                                axis=0)
            a = dot(s, wr_ref[m])
            b = dot(s, wi_ref[m])
            c_scr[:, m, :] = a[:np_] - b[np_:]
            c_scr[:, _MODES + m, :] = b[:np_] + a[np_:]

    @pl.when(i > np_)
    def _inverse():
        p = i - np_ - 1
        c = c_scr[pl.ds(p, 1)][0]
        ch, cl = _split_bf16(c)
        cs = jnp.concatenate([ch, ch, cl], axis=0)
        bic = bic_ref[...]
        out = jax.lax.dot_general(bic, cs, (((1,), (0,)), ((), ())),
                                  preferred_element_type=jnp.float32)
        out_ref[0] = out


def kernel(q, k, v, mask, weights1_real, weights1_imag):
    del k, v, mask
    B, L, H, E = q.shape
    M = _MODES
    NP = B * H

    mi = jnp.arange(M, dtype=jnp.int32)
    li = jnp.arange(L, dtype=jnp.int32)
    phase = (mi[:, None] * li[None, :]) % L
    ang = phase.astype(jnp.float32) * jnp.float32(2.0 * math.pi / L)
    c = jnp.cos(ang)
    s = jnp.sin(ang)
    bf = jnp.concatenate([c, -s], axis=0)
    bf_h, bf_l = _split_bf16(bf)
    bf2 = jnp.concatenate([bf_h, bf_l], axis=0)

    sc = jnp.full((M, 1), 2.0 / L, dtype=jnp.float32).at[0, 0].set(1.0 / L)
    si = jnp.full((M, 1), -1.0 / L, dtype=jnp.float32).at[0, 0].set(0.0)
    bi = jnp.concatenate([sc * c, si * s], axis=0).T
    bi_h, bi_l = _split_bf16(bi)
    bic = jnp.concatenate([bi_h, bi_l, bi_h], axis=1)

    q3 = q.reshape(B, L, H * E)
    last = NP - 1
    out = pl.pallas_call(
        _fca_kernel,
        grid=(2 * NP + 1,),
        in_specs=[
            pl.BlockSpec((1, L, E),
                         lambda i: (jnp.minimum(i, last) // H, 0,
                                    jnp.minimum(i, last) % H)),
            pl.BlockSpec((M, E, E), lambda i: (0, 0, 0)),
            pl.BlockSpec((M, E, E), lambda i: (0, 0, 0)),
            pl.BlockSpec((2 * _NC, L), lambda i: (0, 0)),
            pl.BlockSpec((L, 3 * _NC), lambda i: (0, 0)),
        ],
        out_specs=pl.BlockSpec(
            (1, L, E),
            lambda i: (jnp.clip(i - NP - 1, 0, last) // H, 0,
                       jnp.clip(i - NP - 1, 0, last) % H)),
        out_shape=jax.ShapeDtypeStruct((B, L, H * E), jnp.float32),
        scratch_shapes=[
            pltpu.VMEM((NP, _NC, E), jnp.float32),
            pltpu.VMEM((NP, _NC, E), jnp.float32),
        ],
    )(q3, weights1_real, weights1_imag, bf2, bic)
    return out.reshape(B, L, H, E)
